# baseline (speedup 1.0000x reference)
import jax
import jax.numpy as jnp
from jax.experimental import pallas as pl
from jax.experimental.pallas import tpu as pltpu

N = 2048
IN_C = 512
OC = 512
OCP = 256
M = 128
AROWS = 512
ROWS = 512
F32 = jnp.float32
BF16 = jnp.bfloat16


def _dot(a, b):
    return jnp.dot(a.astype(BF16), b.astype(BF16), preferred_element_type=F32)


def _dot_t(a, b):
    return jax.lax.dot_general(a.astype(BF16), b.astype(BF16),
                               (((0,), (0,)), ((), ())),
                               preferred_element_type=F32)


def _phase_a(xr_ref, xi_ref, qr_ref, qi_ref, w_ref, wl_ref, wres_ref,
             rcol_ref, zc0_ref, zc1_ref, uu_ref, vv_ref, gp_ref, gm_ref):
    k = pl.program_id(0)
    xk_r = xr_ref[...].astype(BF16)
    xk_i = xi_ref[...].astype(BF16)

    w0 = w_ref[0]
    w1 = w_ref[1]
    zc0_ref[...] = jnp.concatenate(
        [_dot(xk_r, w0), _dot(xk_i, w0)], axis=1).astype(BF16)
    zc1_ref[...] = jnp.concatenate(
        [_dot(xk_r, w1), _dot(xk_i, w1)], axis=1).astype(BF16)

    qk_r = qr_ref[...].astype(BF16)
    qk_i = qi_ref[...].astype(BF16)
    gp_k = _dot_t(qk_r, xk_r) + _dot_t(qk_i, xk_i)
    gm_k = _dot_t(qk_i, xk_r) - _dot_t(qk_r, xk_i)

    @pl.when(k == 0)
    def _ginit():
        gp_ref[...] = gp_k
        gm_ref[...] = gm_k

    @pl.when(k > 0)
    def _gacc():
        gp_ref[...] += gp_k
        gm_ref[...] += gm_k

    @pl.when(k == (N // AROWS) - 1)
    def _coeffs():
        rcol = rcol_ref[...]
        t_long = rcol * rcol
        t_res = rcol
        gp = gp_ref[...]
        gm = gm_ref[...]
        u_l = t_long * _dot(gp, wl_ref[...])
        v_l = t_long * _dot(gm, wl_ref[...])
        u_r = t_res * _dot(gp, wres_ref[...])
        v_r = t_res * _dot(gm, wres_ref[...])
        uu_ref[...] = jnp.concatenate(
            [u_r[:, :OCP], u_r[:, OCP:] + u_l], axis=1)
        vv_ref[...] = jnp.concatenate(
            [v_r[:, :OCP], v_r[:, OCP:] + v_l], axis=1)


def _phase_b(lr0_ref, li0_ref, lr1_ref, li1_ref, zc0_ref, zc1_ref,
             qr_ref, qi_ref, uu_ref, vv_ref, bias_ref,
             real_ref, imag_ref):
    i = pl.program_id(0)
    zc0 = zc0_ref[...]
    zc1 = zc1_ref[...]

    p0 = _dot(lr0_ref[...], zc0)
    q0 = _dot(li0_ref[...], zc0)
    p1 = _dot(lr1_ref[...], zc1)
    q1 = _dot(li1_ref[...], zc1)

    dense_real = (p0[:, :OCP] - q0[:, OCP:]) + (p1[:, :OCP] - q1[:, OCP:])
    dense_imag = (q0[:, :OCP] + p0[:, OCP:]) + (q1[:, :OCP] + p1[:, OCP:])

    spec_real = _dot(qr_ref[...], uu_ref[...]) + _dot(qi_ref[...], vv_ref[...])
    spec_imag = _dot(qi_ref[...], uu_ref[...]) - _dot(qr_ref[...], vv_ref[...])

    z0 = zc0_ref[pl.ds(i * ROWS, ROWS), :].astype(F32)
    z1 = zc1_ref[pl.ds(i * ROWS, ROWS), :].astype(F32)
    bias = bias_ref[...]

    real_left = dense_real + spec_real[:, :OCP] + z0[:, :OCP] + bias[:, :OCP]
    real_right = spec_real[:, OCP:] + z1[:, :OCP] + bias[:, OCP:]
    imag_left = dense_imag + spec_imag[:, :OCP] + z0[:, OCP:] + bias[:, :OCP]
    imag_right = spec_imag[:, OCP:] + z1[:, OCP:] + bias[:, OCP:]

    real_ref[...] = jnp.concatenate([real_left, real_right], axis=1)
    imag_ref[...] = jnp.concatenate([imag_left, imag_right], axis=1)


def kernel(X_real, X_imag, L_real_0, L_real_1, L_imag_0, L_imag_1, R,
           Qreal, Qimag, weight, weight_long, weight_res, bias):
    wl = weight_long[0]
    wres = weight_res[0]
    rcol = R.reshape(M, 1)

    arow = pl.BlockSpec((AROWS, IN_C), lambda k: (k, 0))
    aqrow = pl.BlockSpec((AROWS, M), lambda k: (k, 0))
    awhole = lambda s: pl.BlockSpec(s, lambda k: tuple(0 for _ in s))
    azrow = pl.BlockSpec((AROWS, OC), lambda k: (k, 0))

    zc0, zc1, uu, vv = pl.pallas_call(
        _phase_a,
        grid=(N // AROWS,),
        out_shape=(
            jax.ShapeDtypeStruct((N, OC), BF16),
            jax.ShapeDtypeStruct((N, OC), BF16),
            jax.ShapeDtypeStruct((M, OC), F32),
            jax.ShapeDtypeStruct((M, OC), F32),
        ),
        in_specs=[
            arow, arow, aqrow, aqrow,
            awhole((2, IN_C, OCP)), awhole((IN_C, OCP)), awhole((IN_C, OC)),
            awhole((M, 1)),
        ],
        out_specs=(azrow, azrow, awhole((M, OC)), awhole((M, OC))),
        scratch_shapes=[
            pltpu.VMEM((M, OC), F32),
            pltpu.VMEM((M, OC), F32),
        ],
        compiler_params=pltpu.CompilerParams(
            dimension_semantics=("arbitrary",)),
    )(X_real, X_imag, Qreal, Qimag, weight, wl, wres, rcol)

    row = pl.BlockSpec((ROWS, N), lambda i: (i, 0))
    rowq = pl.BlockSpec((ROWS, M), lambda i: (i, 0))
    whole = lambda s: pl.BlockSpec(s, lambda i: (0, 0))
    out_row = pl.BlockSpec((ROWS, OC), lambda i: (i, 0))

    real, imag = pl.pallas_call(
        _phase_b,
        grid=(N // ROWS,),
        out_shape=(
            jax.ShapeDtypeStruct((N, OC), F32),
            jax.ShapeDtypeStruct((N, OC), F32),
        ),
        in_specs=[
            row, row, row, row,
            whole((N, OC)), whole((N, OC)),
            rowq, rowq,
            whole((M, OC)), whole((M, OC)), whole((1, OC)),
        ],
        out_specs=(out_row, out_row),
        compiler_params=pltpu.CompilerParams(
            dimension_semantics=("arbitrary",)),
    )(L_real_0, L_imag_0, L_real_1, L_imag_1, zc0, zc1,
      Qreal, Qimag, uu, vv, bias)

    return (real, imag)


# speedup vs baseline: 1.2030x; 1.2030x over previous
import jax
import jax.numpy as jnp
from jax.experimental import pallas as pl
from jax.experimental.pallas import tpu as pltpu

N = 2048
IN_C = 512
OC = 512
OCP = 256
M = 128
AROWS = 512
ROWS = 256
NA = N // AROWS
NB = N // ROWS
NSLOT = 3
F32 = jnp.float32
BF16 = jnp.bfloat16


def _dot(a, b):
    return jnp.dot(a.astype(BF16), b.astype(BF16), preferred_element_type=F32)


def _dot_t(a, b):
    return jax.lax.dot_general(a.astype(BF16), b.astype(BF16),
                               (((0,), (0,)), ((), ())),
                               preferred_element_type=F32)


def _kernel(xr_ref, xi_ref, qa_r_ref, qa_i_ref, qb_r_ref, qb_i_ref,
            w_ref, wl_ref, wres_ref, rcol_ref, bias_ref,
            lr0_hbm, li0_hbm, lr1_hbm, li1_hbm,
            real_ref, imag_ref,
            zc0_s, zc1_s, gp_s, gm_s, uu_s, vv_s, lbuf, sems):
    s = pl.program_id(0)
    l_hbm = (lr0_hbm, li0_hbm, lr1_hbm, li1_hbm)

    def copies(b, slot):
        rows = pl.ds(b * ROWS, ROWS)
        return [
            pltpu.make_async_copy(ref.at[rows, :], lbuf.at[slot, j],
                                  sems.at[slot, j])
            for j, ref in enumerate(l_hbm)
        ]

    @pl.when(s == 0)
    def _kick_first():
        for c in copies(0, 0):
            c.start()
        for c in copies(1, 1):
            c.start()

    @pl.when(jnp.logical_and(s >= NA, s <= NA + NB - 3))
    def _kick_ahead():
        b = s - 2
        for c in copies(b, b % NSLOT):
            c.start()

    @pl.when(s < NA)
    def _phase_a():
        xk_r = xr_ref[...].astype(BF16)
        xk_i = xi_ref[...].astype(BF16)
        arows = pl.ds(s * AROWS, AROWS)
        w0 = w_ref[0]
        w1 = w_ref[1]
        zc0_s[arows, :] = jnp.concatenate(
            [_dot(xk_r, w0), _dot(xk_i, w0)], axis=1).astype(BF16)
        zc1_s[arows, :] = jnp.concatenate(
            [_dot(xk_r, w1), _dot(xk_i, w1)], axis=1).astype(BF16)

        qk_r = qa_r_ref[...].astype(BF16)
        qk_i = qa_i_ref[...].astype(BF16)
        gp_k = _dot_t(qk_r, xk_r) + _dot_t(qk_i, xk_i)
        gm_k = _dot_t(qk_i, xk_r) - _dot_t(qk_r, xk_i)

        @pl.when(s == 0)
        def _ginit():
            gp_s[...] = gp_k
            gm_s[...] = gm_k

        @pl.when(s > 0)
        def _gacc():
            gp_s[...] += gp_k
            gm_s[...] += gm_k

        @pl.when(s == NA - 1)
        def _coeffs():
            rcol = rcol_ref[...]
            t_long = rcol * rcol
            t_res = rcol
            gp = gp_s[...]
            gm = gm_s[...]
            u_l = t_long * _dot(gp, wl_ref[...])
            v_l = t_long * _dot(gm, wl_ref[...])
            u_r = t_res * _dot(gp, wres_ref[...])
            v_r = t_res * _dot(gm, wres_ref[...])
            uu_s[...] = jnp.concatenate(
                [u_r[:, :OCP], u_r[:, OCP:] + u_l], axis=1)
            vv_s[...] = jnp.concatenate(
                [v_r[:, :OCP], v_r[:, OCP:] + v_l], axis=1)

    @pl.when(s >= NA)
    def _phase_b():
        k = s - NA
        slot = jax.lax.rem(k, NSLOT)
        for c in copies(k, slot):
            c.wait()
        lr0 = lbuf[slot, 0]
        li0 = lbuf[slot, 1]
        lr1 = lbuf[slot, 2]
        li1 = lbuf[slot, 3]

        zc0 = zc0_s[...]
        zc1 = zc1_s[...]
        p0 = _dot(lr0, zc0)
        q0 = _dot(li0, zc0)
        p1 = _dot(lr1, zc1)
        q1 = _dot(li1, zc1)
        dense_real = (p0[:, :OCP] - q0[:, OCP:]) + (p1[:, :OCP] - q1[:, OCP:])
        dense_imag = (q0[:, :OCP] + p0[:, OCP:]) + (q1[:, :OCP] + p1[:, OCP:])

        uu = uu_s[...]
        vv = vv_s[...]
        spec_real = _dot(qb_r_ref[...], uu) + _dot(qb_i_ref[...], vv)
        spec_imag = _dot(qb_i_ref[...], uu) - _dot(qb_r_ref[...], vv)

        rows = pl.ds(k * ROWS, ROWS)
        z0 = zc0_s[rows, :].astype(F32)
        z1 = zc1_s[rows, :].astype(F32)
        bias = bias_ref[...]

        real_l = dense_real + spec_real[:, :OCP] + z0[:, :OCP] + bias[:, :OCP]
        real_r = spec_real[:, OCP:] + z1[:, :OCP] + bias[:, OCP:]
        imag_l = dense_imag + spec_imag[:, :OCP] + z0[:, OCP:] + bias[:, :OCP]
        imag_r = spec_imag[:, OCP:] + z1[:, OCP:] + bias[:, OCP:]

        real_ref[...] = jnp.concatenate([real_l, real_r], axis=1)
        imag_ref[...] = jnp.concatenate([imag_l, imag_r], axis=1)


def kernel(X_real, X_imag, L_real_0, L_real_1, L_imag_0, L_imag_1, R,
           Qreal, Qimag, weight, weight_long, weight_res, bias):
    wl = weight_long[0]
    wres = weight_res[0]
    rcol = R.reshape(M, 1)

    a_idx = lambda s: (jnp.minimum(s, NA - 1), 0)
    b_idx = lambda s: (jnp.clip(s - NA, 0, NB - 1), 0)
    whole = lambda shp: pl.BlockSpec(shp, lambda s: tuple(0 for _ in shp))
    hbm = pl.BlockSpec(memory_space=pl.ANY)

    real, imag = pl.pallas_call(
        _kernel,
        grid=(NA + NB,),
        out_shape=(
            jax.ShapeDtypeStruct((N, OC), F32),
            jax.ShapeDtypeStruct((N, OC), F32),
        ),
        in_specs=[
            pl.BlockSpec((AROWS, IN_C), a_idx),
            pl.BlockSpec((AROWS, IN_C), a_idx),
            pl.BlockSpec((AROWS, M), a_idx),
            pl.BlockSpec((AROWS, M), a_idx),
            pl.BlockSpec((ROWS, M), b_idx),
            pl.BlockSpec((ROWS, M), b_idx),
            whole((2, IN_C, OCP)),
            whole((IN_C, OCP)),
            whole((IN_C, OC)),
            whole((M, 1)),
            whole((1, OC)),
            hbm, hbm, hbm, hbm,
        ],
        out_specs=(
            pl.BlockSpec((ROWS, OC), b_idx),
            pl.BlockSpec((ROWS, OC), b_idx),
        ),
        scratch_shapes=[
            pltpu.VMEM((N, OC), BF16),
            pltpu.VMEM((N, OC), BF16),
            pltpu.VMEM((M, OC), F32),
            pltpu.VMEM((M, OC), F32),
            pltpu.VMEM((M, OC), F32),
            pltpu.VMEM((M, OC), F32),
            pltpu.VMEM((NSLOT, 4, ROWS, N), F32),
            pltpu.SemaphoreType.DMA((NSLOT, 4)),
        ],
        compiler_params=pltpu.CompilerParams(
            dimension_semantics=("arbitrary",)),
    )(X_real, X_imag, Qreal, Qimag, Qreal, Qimag,
      weight, wl, wres, rcol, bias,
      L_real_0, L_imag_0, L_real_1, L_imag_1)

    return (real, imag)
